# initial kernel scaffold (unmeasured)
import jax
import jax.numpy as jnp
from jax import lax
from jax.experimental import pallas as pl
from jax.experimental.pallas import tpu as pltpu


def kernel(
    t,
):
    def body(*refs):
        pass

    out_shape = jax.ShapeDtypeStruct(..., jnp.float32)
    return pl.pallas_call(body, out_shape=out_shape)(...)


# baseline (device time: 290217 ns/iter reference)
import jax
import jax.numpy as jnp
from jax import lax
from jax.experimental import pallas as pl
from jax.experimental.pallas import tpu as pltpu

N_DEV = 4


def kernel(t):
    m, n = t.shape

    def body(x_ref, out_ref, comm_ref, send_sems, recv_sems):
        my = lax.axis_index("i")
        left = lax.rem(my - 1 + N_DEV, N_DEV)
        right = lax.rem(my + 1, N_DEV)

        barrier_sem = pltpu.get_barrier_semaphore()
        for nbr in [left, right]:
            pl.semaphore_signal(
                barrier_sem, inc=1,
                device_id=(nbr,), device_id_type=pl.DeviceIdType.MESH,
            )
        pl.semaphore_wait(barrier_sem, 2)

        out_ref[:, :] = x_ref[:, :]
        comm_ref[0, :, :] = x_ref[:, :]

        for h in range(N_DEV - 1):
            send_slot = h % 2
            recv_slot = (h + 1) % 2
            rdma = pltpu.make_async_remote_copy(
                src_ref=comm_ref.at[send_slot],
                dst_ref=comm_ref.at[recv_slot],
                send_sem=send_sems.at[send_slot],
                recv_sem=recv_sems.at[recv_slot],
                device_id=(right,),
                device_id_type=pl.DeviceIdType.MESH,
            )
            rdma.start()
            rdma.wait()
            out_ref[:, :] = out_ref[:, :] + comm_ref[recv_slot, :, :]

        s = out_ref[:, :]
        r = jnp.maximum(s, 0.0)
        out_ref[:, :] = jnp.tanh(s) * s * s + r * r * r

    return pl.pallas_call(
        body,
        out_shape=jax.ShapeDtypeStruct((m, n), jnp.float32),
        in_specs=[pl.BlockSpec(memory_space=pltpu.VMEM)],
        out_specs=pl.BlockSpec(memory_space=pltpu.VMEM),
        scratch_shapes=[
            pltpu.VMEM((2, m, n), jnp.float32),
            pltpu.SemaphoreType.DMA((2,)),
            pltpu.SemaphoreType.DMA((2,)),
        ],
        compiler_params=pltpu.CompilerParams(collective_id=0),
    )(t)


# device time: 88122 ns/iter; 3.2934x vs baseline; 3.2934x over previous
import jax
import jax.numpy as jnp
from jax import lax
from jax.experimental import pallas as pl
from jax.experimental.pallas import tpu as pltpu

N_DEV = 4
CH = 256
HALF = N_DEV * CH


def _f(s):
    r = jnp.maximum(s, 0.0)
    return jnp.tanh(s) * s * s + r * r * r


def kernel(t):
    m, n = t.shape
    assert m == 2 * HALF

    def body(x_ref, out_ref, rs_f, rs_r, ssem, rsem):
        my = lax.axis_index("i")
        right = lax.rem(my + 1, N_DEV)
        left = lax.rem(my + 3, N_DEV)

        barrier_sem = pltpu.get_barrier_semaphore()
        for nbr in [left, right]:
            pl.semaphore_signal(
                barrier_sem, inc=1,
                device_id=(nbr,), device_id_type=pl.DeviceIdType.MESH,
            )
        pl.semaphore_wait(barrier_sem, 2)

        def fwd_rows(q):
            return pl.ds(q * CH, CH)

        def rev_rows(q):
            return pl.ds(HALF + q * CH, CH)

        def rdma(src, dst, d, dev, h):
            return pltpu.make_async_remote_copy(
                src_ref=src, dst_ref=dst,
                send_sem=ssem.at[d, h], recv_sem=rsem.at[d, h],
                device_id=(dev,), device_id_type=pl.DeviceIdType.MESH,
            )

        for h in range(N_DEV - 1):
            cf = lax.rem(my - h + N_DEV, N_DEV)
            cr = lax.rem(my + h, N_DEV)
            if h == 0:
                src_f = x_ref.at[fwd_rows(cf)]
                src_r = x_ref.at[rev_rows(cr)]
            else:
                rs_f[h - 1, :, :] = rs_f[h - 1, :, :] + x_ref[fwd_rows(cf), :]
                rs_r[h - 1, :, :] = rs_r[h - 1, :, :] + x_ref[rev_rows(cr), :]
                src_f = rs_f.at[h - 1]
                src_r = rs_r.at[h - 1]
            rf = rdma(src_f, rs_f.at[h], 0, right, h)
            rr = rdma(src_r, rs_r.at[h], 1, left, h)
            rf.start()
            rr.start()
            rf.wait()
            rr.wait()

        of = lax.rem(my + 1, N_DEV)
        orv = lax.rem(my + N_DEV - 1, N_DEV)
        out_ref[fwd_rows(of), :] = _f(rs_f[2, :, :] + x_ref[fwd_rows(of), :])
        out_ref[rev_rows(orv), :] = _f(rs_r[2, :, :] + x_ref[rev_rows(orv), :])

        for h in range(N_DEV - 1):
            cf = lax.rem(my + 1 - h + N_DEV, N_DEV)
            cr = lax.rem(my - 1 + h + N_DEV, N_DEV)
            rf = rdma(out_ref.at[fwd_rows(cf)], out_ref.at[fwd_rows(cf)],
                      0, right, 3 + h)
            rr = rdma(out_ref.at[rev_rows(cr)], out_ref.at[rev_rows(cr)],
                      1, left, 3 + h)
            rf.start()
            rr.start()
            rf.wait()
            rr.wait()

    return pl.pallas_call(
        body,
        out_shape=jax.ShapeDtypeStruct((m, n), jnp.float32),
        in_specs=[pl.BlockSpec(memory_space=pltpu.VMEM)],
        out_specs=pl.BlockSpec(memory_space=pltpu.VMEM),
        scratch_shapes=[
            pltpu.VMEM((N_DEV - 1, CH, n), jnp.float32),
            pltpu.VMEM((N_DEV - 1, CH, n), jnp.float32),
            pltpu.SemaphoreType.DMA((2, 6)),
            pltpu.SemaphoreType.DMA((2, 6)),
        ],
        compiler_params=pltpu.CompilerParams(collective_id=0),
    )(t)


# device time: 79394 ns/iter; 3.6554x vs baseline; 1.1099x over previous
import jax
import jax.numpy as jnp
from jax import lax
from jax.experimental import pallas as pl
from jax.experimental.pallas import tpu as pltpu

N_DEV = 4
CH = 256
NSUB = 2
SUB = CH // NSUB
HALF = N_DEV * CH
NHOP = 2 * (N_DEV - 1)


def _f(s):
    r = jnp.maximum(s, 0.0)
    return jnp.tanh(s) * s * s + r * r * r


def kernel(t):
    m, n = t.shape
    assert m == 2 * HALF

    def body(x_ref, out_ref, rs_f, rs_r, ssem, rsem):
        my = lax.axis_index("i")
        right = lax.rem(my + 1, N_DEV)
        left = lax.rem(my + 3, N_DEV)

        barrier_sem = pltpu.get_barrier_semaphore()
        for nbr in [left, right]:
            pl.semaphore_signal(
                barrier_sem, inc=1,
                device_id=(nbr,), device_id_type=pl.DeviceIdType.MESH,
            )
        pl.semaphore_wait(barrier_sem, 2)

        def rows(d, q, s):
            return pl.ds(d * HALF + q * CH + s * SUB, SUB)

        def rdma(src, dst, d, s, h, dev):
            return pltpu.make_async_remote_copy(
                src_ref=src, dst_ref=dst,
                send_sem=ssem.at[d, s, h], recv_sem=rsem.at[d, s, h],
                device_id=(dev,), device_id_type=pl.DeviceIdType.MESH,
            )

        dev_of = {0: right, 1: left}
        rs_of = {0: rs_f, 1: rs_r}
        def chunk_id(d, h):
            if h < N_DEV - 1:
                delta = -h if d == 0 else h
            else:
                ha = h - (N_DEV - 1)
                delta = 1 - ha if d == 0 else -1 + ha
            return lax.rem(my + delta + 2 * N_DEV, N_DEV)

        started = {}
        order = [(0, 0), (1, 0), (0, 1), (1, 1)]

        for h in range(N_DEV - 1):
            for d, s in order:
                c = chunk_id(d, h)
                if h == 0:
                    src = x_ref.at[rows(d, c, s)]
                else:
                    started[(d, s, h - 1)].wait_recv()
                    buf = rs_of[d]
                    sub = pl.ds(s * SUB, SUB)
                    buf[h - 1, sub, :] = buf[h - 1, sub, :] + x_ref[rows(d, c, s), :]
                    src = buf.at[h - 1, sub]
                r = rdma(src, rs_of[d].at[h, pl.ds(s * SUB, SUB)], d, s, h, dev_of[d])
                r.start()
                started[(d, s, h)] = r

        for d, s in order:
            started[(d, s, N_DEV - 2)].wait_recv()
            c = chunk_id(d, N_DEV - 1)
            sub = pl.ds(s * SUB, SUB)
            out_ref[rows(d, c, s), :] = _f(
                rs_of[d][N_DEV - 2, sub, :] + x_ref[rows(d, c, s), :]
            )
            r = rdma(out_ref.at[rows(d, c, s)], out_ref.at[rows(d, c, s)],
                     d, s, N_DEV - 1, dev_of[d])
            r.start()
            started[(d, s, N_DEV - 1)] = r

        for h in range(N_DEV, NHOP):
            for d, s in order:
                started[(d, s, h - 1)].wait_recv()
                c = chunk_id(d, h)
                r = rdma(out_ref.at[rows(d, c, s)], out_ref.at[rows(d, c, s)],
                         d, s, h, dev_of[d])
                r.start()
                started[(d, s, h)] = r

        for d, s in order:
            started[(d, s, NHOP - 1)].wait_recv()
        for r in started.values():
            r.wait_send()

    return pl.pallas_call(
        body,
        out_shape=jax.ShapeDtypeStruct((m, n), jnp.float32),
        in_specs=[pl.BlockSpec(memory_space=pltpu.VMEM)],
        out_specs=pl.BlockSpec(memory_space=pltpu.VMEM),
        scratch_shapes=[
            pltpu.VMEM((N_DEV - 1, CH, n), jnp.float32),
            pltpu.VMEM((N_DEV - 1, CH, n), jnp.float32),
            pltpu.SemaphoreType.DMA((2, NSUB, NHOP)),
            pltpu.SemaphoreType.DMA((2, NSUB, NHOP)),
        ],
        compiler_params=pltpu.CompilerParams(collective_id=0),
    )(t)
